# Initial kernel scaffold; baseline (speedup 1.0000x reference)
import jax
import jax.numpy as jnp
from jax.experimental import pallas as pl


def kernel(node_ids, cat_sector, cat_region, cat_venue, id_table, sector_table, region_table, venue_table):
    raise NotImplementedError("write your pallas kernel here")


# speedup vs baseline: 4.3187x; 4.3187x over previous
import jax
import jax.numpy as jnp
from jax import lax
from jax.experimental import pallas as pl
from jax.experimental.pallas import tpu as pltpu
from jax.experimental.pallas import tpu_sc as plsc

NC, NS = 2, 16
NW = NC * NS
B, L = 16384, 20
R = B * L
CHUNK = 128
PER_W = R // NW
N_CHUNKS = PER_W // CHUNK

D_ID = 32
D_CAT = 16
D_OUT = D_ID + 3 * D_CAT


def _emb_body(nid, sec, reg, ven, id_t, sec_t, reg_t, ven_t, out,
              nid_v, sec_v, reg_v, ven_v, id_r, sec_r, reg_r, ven_r, sem):
    wid = lax.axis_index("s") * NC + lax.axis_index("c")
    base_w = wid * PER_W

    def chunk(i, carry):
        base = base_w + i * CHUNK
        pltpu.sync_copy(nid.at[pl.ds(base, CHUNK)], nid_v)
        pltpu.sync_copy(sec.at[pl.ds(base, CHUNK)], sec_v)
        pltpu.sync_copy(reg.at[pl.ds(base, CHUNK)], reg_v)
        pltpu.sync_copy(ven.at[pl.ds(base, CHUNK)], ven_v)
        c1 = pltpu.async_copy(id_t.at[nid_v], id_r, sem)
        c2 = pltpu.async_copy(sec_t.at[sec_v], sec_r, sem)
        c3 = pltpu.async_copy(reg_t.at[reg_v], reg_r, sem)
        c4 = pltpu.async_copy(ven_t.at[ven_v], ven_r, sem)
        c1.wait()
        c2.wait()
        c3.wait()
        c4.wait()
        pltpu.sync_copy(id_r, out.at[pl.ds(base, CHUNK), pl.ds(0, D_ID)])
        pltpu.sync_copy(sec_r, out.at[pl.ds(base, CHUNK), pl.ds(D_ID, D_CAT)])
        pltpu.sync_copy(reg_r, out.at[pl.ds(base, CHUNK), pl.ds(D_ID + D_CAT, D_CAT)])
        pltpu.sync_copy(ven_r, out.at[pl.ds(base, CHUNK), pl.ds(D_ID + 2 * D_CAT, D_CAT)])
        return carry

    lax.fori_loop(0, N_CHUNKS, chunk, 0)


def kernel(node_ids, cat_sector, cat_region, cat_venue,
           id_table, sector_table, region_table, venue_table):
    nid = node_ids.reshape(-1).astype(jnp.int32)
    sec = cat_sector.reshape(-1).astype(jnp.int32)
    reg = cat_region.reshape(-1).astype(jnp.int32)
    ven = cat_venue.reshape(-1).astype(jnp.int32)

    call = pl.kernel(
        _emb_body,
        out_type=jax.ShapeDtypeStruct((R, D_OUT), jnp.float32),
        mesh=plsc.VectorSubcoreMesh(
            core_axis_name="c", subcore_axis_name="s",
            num_cores=NC, num_subcores=NS),
        scratch_types=[
            pltpu.VMEM((CHUNK,), jnp.int32),
            pltpu.VMEM((CHUNK,), jnp.int32),
            pltpu.VMEM((CHUNK,), jnp.int32),
            pltpu.VMEM((CHUNK,), jnp.int32),
            pltpu.VMEM((CHUNK, D_ID), jnp.float32),
            pltpu.VMEM((CHUNK, D_CAT), jnp.float32),
            pltpu.VMEM((CHUNK, D_CAT), jnp.float32),
            pltpu.VMEM((CHUNK, D_CAT), jnp.float32),
            pltpu.SemaphoreType.DMA,
        ],
        compiler_params=pltpu.CompilerParams(use_tc_tiling_on_sc=False),
    )
    out = call(nid, sec, reg, ven, id_table, sector_table, region_table,
               venue_table)
    return out.reshape(node_ids.shape[0], node_ids.shape[1], D_OUT)


# speedup vs baseline: 4.7256x; 1.0942x over previous
import jax
import jax.numpy as jnp
from jax import lax
from jax.experimental import pallas as pl
from jax.experimental.pallas import tpu as pltpu
from jax.experimental.pallas import tpu_sc as plsc

NC, NS = 2, 16
NW = NC * NS
B, L = 16384, 20
R = B * L
CHUNK = 128
NBUF = 4
PER_W = R // NW
N_CHUNKS = PER_W // CHUNK

D_ID = 32
D_CAT = 16
D_OUT = D_ID + 3 * D_CAT


def _emb_body(nid, sec, reg, ven, id_t, sec_t, reg_t, ven_t, out,
              idx_v, id_r, sec_r, reg_r, ven_r,
              sem_i, sem_g, sem_w):
    wid = lax.axis_index("s") * NC + lax.axis_index("c")
    base_w = wid * PER_W

    def idx_copies(base, b):
        return [
            pltpu.make_async_copy(nid.at[pl.ds(base, CHUNK)], idx_v.at[b, 0],
                                  sem_i.at[b]),
            pltpu.make_async_copy(sec.at[pl.ds(base, CHUNK)], idx_v.at[b, 1],
                                  sem_i.at[b]),
            pltpu.make_async_copy(reg.at[pl.ds(base, CHUNK)], idx_v.at[b, 2],
                                  sem_i.at[b]),
            pltpu.make_async_copy(ven.at[pl.ds(base, CHUNK)], idx_v.at[b, 3],
                                  sem_i.at[b]),
        ]

    def gather_copies(b):
        return [
            pltpu.make_async_copy(id_t.at[idx_v.at[b, 0]], id_r.at[b],
                                  sem_g.at[b]),
            pltpu.make_async_copy(sec_t.at[idx_v.at[b, 1]], sec_r.at[b],
                                  sem_g.at[b]),
            pltpu.make_async_copy(reg_t.at[idx_v.at[b, 2]], reg_r.at[b],
                                  sem_g.at[b]),
            pltpu.make_async_copy(ven_t.at[idx_v.at[b, 3]], ven_r.at[b],
                                  sem_g.at[b]),
        ]

    def write_copies(base, b):
        rows = pl.ds(base, CHUNK)
        return [
            pltpu.make_async_copy(id_r.at[b], out.at[rows, pl.ds(0, D_ID)],
                                  sem_w.at[b]),
            pltpu.make_async_copy(sec_r.at[b], out.at[rows, pl.ds(D_ID, D_CAT)],
                                  sem_w.at[b]),
            pltpu.make_async_copy(reg_r.at[b],
                                  out.at[rows, pl.ds(D_ID + D_CAT, D_CAT)],
                                  sem_w.at[b]),
            pltpu.make_async_copy(ven_r.at[b],
                                  out.at[rows, pl.ds(D_ID + 2 * D_CAT, D_CAT)],
                                  sem_w.at[b]),
        ]

    def outer(g, carry):
        for b in range(NBUF):
            base = base_w + (g * NBUF + b) * CHUNK

            @pl.when(g > 0)
            def _():
                for c in write_copies(base, b):
                    c.wait()

            for c in idx_copies(base, b):
                c.start()

        for b in range(NBUF):
            for c in idx_copies(base_w + (g * NBUF + b) * CHUNK, b):
                c.wait()
            for c in gather_copies(b):
                c.start()

        for b in range(NBUF):
            for c in gather_copies(b):
                c.wait()
            for c in write_copies(base_w + (g * NBUF + b) * CHUNK, b):
                c.start()

        return carry

    lax.fori_loop(0, N_CHUNKS // NBUF, outer, 0, unroll=False)

    for b in range(NBUF):
        for c in write_copies(base_w, b):
            c.wait()


def kernel(node_ids, cat_sector, cat_region, cat_venue,
           id_table, sector_table, region_table, venue_table):
    nid = node_ids.reshape(-1).astype(jnp.int32)
    sec = cat_sector.reshape(-1).astype(jnp.int32)
    reg = cat_region.reshape(-1).astype(jnp.int32)
    ven = cat_venue.reshape(-1).astype(jnp.int32)

    call = pl.kernel(
        _emb_body,
        out_type=jax.ShapeDtypeStruct((R, D_OUT), jnp.float32),
        mesh=plsc.VectorSubcoreMesh(
            core_axis_name="c", subcore_axis_name="s",
            num_cores=NC, num_subcores=NS),
        scratch_types=[
            pltpu.VMEM((NBUF, 4, CHUNK), jnp.int32),
            pltpu.VMEM((NBUF, CHUNK, D_ID), jnp.float32),
            pltpu.VMEM((NBUF, CHUNK, D_CAT), jnp.float32),
            pltpu.VMEM((NBUF, CHUNK, D_CAT), jnp.float32),
            pltpu.VMEM((NBUF, CHUNK, D_CAT), jnp.float32),
            pltpu.SemaphoreType.DMA((NBUF,)),
            pltpu.SemaphoreType.DMA((NBUF,)),
            pltpu.SemaphoreType.DMA((NBUF,)),
        ],
        compiler_params=pltpu.CompilerParams(use_tc_tiling_on_sc=False),
    )
    out = call(nid, sec, reg, ven, id_table, sector_table, region_table,
               venue_table)
    return out.reshape(node_ids.shape[0], node_ids.shape[1], D_OUT)
